# Initial kernel scaffold; baseline (speedup 1.0000x reference)
import jax
import jax.numpy as jnp
from jax.experimental import pallas as pl


def kernel(x, prom_x, edge_attr, edge_index, batch, params):
    raise NotImplementedError("write your pallas kernel here")


# speedup vs baseline: 1.6795x; 1.6795x over previous
import functools

import jax
import jax.numpy as jnp
import numpy as np
from jax.experimental import pallas as pl

N = 100000
E = 1600000
B = 500
L = 200
NUMCHIP = 18
HID = 20
HEADS = 4
DH = 5
NEDGE = 2


def _pe_table():
    pos = jnp.arange(L, dtype=jnp.float32)[:, None]
    div = jnp.exp(jnp.arange(0, HID, 2, dtype=jnp.float32) * (-np.log(10000.0) / HID))
    pe = jnp.zeros((L, HID), jnp.float32)
    pe = pe.at[:, 0::2].set(jnp.sin(pos * div))
    pe = pe.at[:, 1::2].set(jnp.cos(pos * div))
    return pe


def _emb_block_kernel(x_ref, w_ref, b_ref, g_ref, beta_ref, o_ref):
    h = jnp.dot(x_ref[...], w_ref[...], preferred_element_type=jnp.float32)
    o_ref[...] = jax.nn.relu(g_ref[...] * (h + b_ref[...]) + beta_ref[...])


def _emb_pallas(x, p):
    n, din = x.shape
    dout = p['W'].shape[1]
    blk = 10000
    assert n % blk == 0
    return pl.pallas_call(
        _emb_block_kernel,
        grid=(n // blk,),
        in_specs=[
            pl.BlockSpec((blk, din), lambda i: (i, 0)),
            pl.BlockSpec((din, dout), lambda i: (0, 0)),
            pl.BlockSpec((1, dout), lambda i: (0, 0)),
            pl.BlockSpec((1, dout), lambda i: (0, 0)),
            pl.BlockSpec((1, dout), lambda i: (0, 0)),
        ],
        out_specs=pl.BlockSpec((blk, dout), lambda i: (i, 0)),
        out_shape=jax.ShapeDtypeStruct((n, dout), jnp.float32),
    )(x, p['W'], p['b'][None, :], p['g'][None, :], p['beta'][None, :])


def _emb_small(h, p):
    return jax.nn.relu(p['g'] * (h @ p['W'] + p['b']) + p['beta'])


def _conv(h, e, src, dst, p):
    hs = h @ p['W_src']
    hd = h @ p['W_dst']
    he = e @ p['W_edge']
    m = (hs[src] + hd[dst] + he).reshape(-1, HEADS, DH)
    lr = jax.nn.leaky_relu(m, 0.2)
    logits = jnp.einsum('ehd,hd->eh', lr, p['att'])
    ex = jnp.exp(logits)
    den = jax.ops.segment_sum(ex, dst, num_segments=N)
    num = jax.ops.segment_sum(
        (hs[src].reshape(-1, HEADS, DH) * ex[:, :, None]).reshape(-1, HID),
        dst, num_segments=N)
    agg = num / (den + 1e-16).reshape(N, HEADS, 1).repeat(DH, 2).reshape(N, HID)
    h_out = jax.nn.relu(h + agg + p['b_node'])
    e_out = jax.nn.relu(jnp.concatenate([h[src], h[dst], e], axis=-1) @ p['We'] + p['be'])
    return h_out, e_out


def kernel(x, prom_x, edge_attr, edge_index, batch, params):
    x = jnp.nan_to_num(x)
    prom_x = jnp.nan_to_num(prom_x.reshape(-1, NUMCHIP))
    edge_attr = jnp.nan_to_num(edge_attr)
    h = _emb_pallas(x, params['emb0'])
    pp = _emb_small(prom_x, params['emb0'])
    h = h + _emb_pallas(h, params['emb1'])
    pp = pp + _emb_small(pp, params['emb1'])
    pe = _pe_table()
    h = h + pe[jnp.arange(N) % L]
    src = edge_index[0]
    dst = edge_index[1]
    e = edge_attr
    h, e = _conv(h, e, src, dst, params['enc'])
    for _ in range(4):
        h, e = _conv(h, e, src, dst, params['encdec'])
    mid = h.reshape(B, L, HID)[:, L // 2, :]
    z = mid + pp
    for lp in params['fc']:
        z = jax.nn.relu(lp['g'] * (z @ lp['W'] + lp['b']) + lp['beta'])
    return z @ params['readout']['W'] + params['readout']['b']


# speedup vs baseline: 3.5692x; 2.1251x over previous
import functools

import jax
import jax.numpy as jnp
import numpy as np
from jax import lax
from jax.experimental import pallas as pl
from jax.experimental.pallas import tpu as pltpu
from jax.experimental.pallas import tpu_sc as plsc

N = 100000
E = 1600000
B = 500
L = 200
NUMCHIP = 18
HID = 20
HEADS = 4
DH = 5
NEDGE = 2

SC_CORES = 2
SC_TILES = 16
WORKERS = SC_CORES * SC_TILES
HALF = N // 2
TPT = 3128
SHROWS = TPT * SC_TILES
BK = 1000
EPT32 = E // WORKERS
EPT16 = E // SC_TILES
PKW = 32
AW = 16
BW = 8


def _pe_table():
    pos = jnp.arange(L, dtype=jnp.float32)[:, None]
    div = jnp.exp(jnp.arange(0, HID, 2, dtype=jnp.float32) * (-np.log(10000.0) / HID))
    pe = jnp.zeros((L, HID), jnp.float32)
    pe = pe.at[:, 0::2].set(jnp.sin(pos * div))
    pe = pe.at[:, 1::2].set(jnp.cos(pos * div))
    return pe


def _sc_gather_body(hsx, hdx, src, dst, hsg, hdg,
                    src_v, dst_v, hs_v, hd_v, sem, sem2):
    cid = lax.axis_index("c")
    sid = lax.axis_index("s")
    wid = cid * SC_TILES + sid

    def block(bi, carry):
        base = wid * EPT32 + bi * BK
        pltpu.sync_copy(src.at[pl.ds(base, BK)], src_v)
        pltpu.sync_copy(dst.at[pl.ds(base, BK)], dst_v)
        cp1 = pltpu.async_copy(hsx.at[src_v], hs_v, sem)
        cp2 = pltpu.async_copy(hdx.at[dst_v], hd_v, sem2)
        cp1.wait()
        pltpu.sync_copy(hs_v, hsg.at[pl.ds(base, BK)])
        cp2.wait()
        pltpu.sync_copy(hd_v, hdg.at[pl.ds(base, BK)])
        return carry

    lax.fori_loop(0, EPT32 // BK, block, 0)


def _sc_gather(hsx, hdx, src, dst):
    mesh = plsc.VectorSubcoreMesh(core_axis_name="c", subcore_axis_name="s",
                                  num_cores=SC_CORES, num_subcores=SC_TILES)
    return pl.kernel(
        _sc_gather_body,
        out_type=[jax.ShapeDtypeStruct((E, PKW), jnp.float32),
                  jax.ShapeDtypeStruct((E, PKW), jnp.float32)],
        mesh=mesh,
        compiler_params=pltpu.CompilerParams(use_tc_tiling_on_sc=False),
        scratch_types=[
            pltpu.VMEM((BK,), jnp.int32),
            pltpu.VMEM((BK,), jnp.int32),
            pltpu.VMEM((BK, PKW), jnp.float32),
            pltpu.VMEM((BK, PKW), jnp.float32),
            pltpu.SemaphoreType.DMA,
            pltpu.SemaphoreType.DMA,
        ],
    )(hsx, hdx, src, dst)


def _sc_scatter_body(vals_a, vals_b, dst, zeros_half, zeros_b,
                     acc_a0, acc_a1, acc_b0, acc_b1,
                     va_v, vb_v, dst_v, lidx_v, acca_sh, accb_sh, sem):
    cid = lax.axis_index("c")
    sid = lax.axis_index("s")
    lo = cid * HALF

    pltpu.sync_copy(zeros_half.at[pl.ds(sid * TPT, TPT)],
                    acca_sh.at[pl.ds(sid * TPT, TPT)])
    pltpu.sync_copy(zeros_b.at[pl.ds(sid * TPT, TPT)],
                    accb_sh.at[pl.ds(sid * TPT, TPT)])
    plsc.subcore_barrier()

    def block(bi, carry):
        base = sid * EPT16 + bi * BK
        pltpu.sync_copy(vals_a.at[pl.ds(base, BK)], va_v)
        pltpu.sync_copy(vals_b.at[pl.ds(base, BK)], vb_v)
        pltpu.sync_copy(dst.at[pl.ds(base, BK)], dst_v)

        def group(j, c2):
            s16 = pl.ds(j * 16, 16)
            d = dst_v[s16]
            li = d - lo
            ok = (li >= 0) & (li < HALF)
            lidx_v[s16] = jnp.where(ok, li, HALF)
            return c2

        lax.fori_loop(0, BK // 16, group, 0)
        pltpu.sync_copy(va_v, acca_sh.at[lidx_v], add=True)
        pltpu.sync_copy(vb_v, accb_sh.at[lidx_v], add=True)
        return carry

    lax.fori_loop(0, EPT16 // BK, block, 0)
    plsc.subcore_barrier()

    @pl.when(cid == 0)
    def _():
        pltpu.sync_copy(acca_sh.at[pl.ds(sid * TPT, TPT)],
                        acc_a0.at[pl.ds(sid * TPT, TPT)])
        pltpu.sync_copy(accb_sh.at[pl.ds(sid * TPT, TPT)],
                        acc_b0.at[pl.ds(sid * TPT, TPT)])

    @pl.when(cid == 1)
    def _():
        pltpu.sync_copy(acca_sh.at[pl.ds(sid * TPT, TPT)],
                        acc_a1.at[pl.ds(sid * TPT, TPT)])
        pltpu.sync_copy(accb_sh.at[pl.ds(sid * TPT, TPT)],
                        acc_b1.at[pl.ds(sid * TPT, TPT)])


def _sc_scatter(vals_a, vals_b, dst, zeros_half, zeros_b):
    mesh = plsc.VectorSubcoreMesh(core_axis_name="c", subcore_axis_name="s",
                                  num_cores=SC_CORES, num_subcores=SC_TILES)
    return pl.kernel(
        _sc_scatter_body,
        out_type=[jax.ShapeDtypeStruct((SHROWS, AW), jnp.float32),
                  jax.ShapeDtypeStruct((SHROWS, AW), jnp.float32),
                  jax.ShapeDtypeStruct((SHROWS, BW), jnp.float32),
                  jax.ShapeDtypeStruct((SHROWS, BW), jnp.float32)],
        mesh=mesh,
        compiler_params=pltpu.CompilerParams(use_tc_tiling_on_sc=False),
        scratch_types=[
            pltpu.VMEM((BK, AW), jnp.float32),
            pltpu.VMEM((BK, BW), jnp.float32),
            pltpu.VMEM((BK,), jnp.int32),
            pltpu.VMEM((BK,), jnp.int32),
            pltpu.VMEM_SHARED((SHROWS, AW), jnp.float32),
            pltpu.VMEM_SHARED((SHROWS, BW), jnp.float32),
            pltpu.SemaphoreType.DMA,
        ],
    )(vals_a, vals_b, dst, zeros_half, zeros_b)


NBLK_TC = 50
BLK = N // NBLK_TC


def _full_specs(arrs):
    return [pl.BlockSpec(a.shape, functools.partial(
        lambda nd, i: (0,) * nd, a.ndim)) for a in arrs]


def _row(v):
    return v.reshape(1, -1)


def _pack_tables(h, wsx, wdx):
    hsx = jnp.dot(h, wsx, preferred_element_type=jnp.float32)
    hdx = jnp.dot(h, wdx, preferred_element_type=jnp.float32)
    return hsx, hdx


def _front_body(x_ref, pe_ref, w0, b0, g0, t0, w1, b1, g1, t1,
                wsx, wdx, h_ref, hsx_ref, hdx_ref):
    xc = jnp.nan_to_num(x_ref[...])
    h0 = jax.nn.relu(g0[...] * (jnp.dot(xc, w0[...],
                                        preferred_element_type=jnp.float32)
                                + b0[...]) + t0[...])
    h1 = jax.nn.relu(g1[...] * (jnp.dot(h0, w1[...],
                                        preferred_element_type=jnp.float32)
                                + b1[...]) + t1[...])
    h = h0 + h1 + jnp.tile(pe_ref[...], (BLK // L, 1))
    h_ref[...] = h
    hsx_ref[...], hdx_ref[...] = _pack_tables(h, wsx[...], wdx[...])


def _front(x, pe, p0, p1, pc):
    consts = [p0['W'], _row(p0['b']), _row(p0['g']), _row(p0['beta']),
              p1['W'], _row(p1['b']), _row(p1['g']), _row(p1['beta']),
              pc['wsx'], pc['wdx']]
    f = pl.pallas_call(
        _front_body,
        grid=(NBLK_TC,),
        in_specs=[
            pl.BlockSpec((BLK, NUMCHIP), lambda i: (i, 0)),
            pl.BlockSpec((L, HID), lambda i: (0, 0)),
        ] + _full_specs(consts),
        out_specs=[
            pl.BlockSpec((BLK, HID), lambda i: (i, 0)),
            pl.BlockSpec((BLK, PKW), lambda i: (i, 0)),
            pl.BlockSpec((BLK, PKW), lambda i: (i, 0)),
        ],
        out_shape=[
            jax.ShapeDtypeStruct((N, HID), jnp.float32),
            jax.ShapeDtypeStruct((N, PKW), jnp.float32),
            jax.ShapeDtypeStruct((N, PKW), jnp.float32),
        ],
    )
    return f(x, pe, *consts)


def _agg_from_acc(acca, accb):
    den = acca[:, :HEADS]
    num = jnp.concatenate([acca[:, HEADS:], accb[:, :HID - (AW - HEADS)]],
                          axis=1)
    denx = jnp.repeat(den + 1e-16, DH, axis=1)
    return num / denx


def _tca_body(h_ref, acca_ref, accb_ref, bn, wsx, wdx,
              hn_ref, hsx_ref, hdx_ref):
    agg = _agg_from_acc(acca_ref[...], accb_ref[...])
    hn = jax.nn.relu(h_ref[...] + agg + bn[...])
    hn_ref[...] = hn
    hsx_ref[...], hdx_ref[...] = _pack_tables(hn, wsx[...], wdx[...])


def _tca(h, acca, accb, bnode, pc):
    consts = [_row(bnode), pc['wsx'], pc['wdx']]
    f = pl.pallas_call(
        _tca_body,
        grid=(NBLK_TC,),
        in_specs=[
            pl.BlockSpec((BLK, HID), lambda i: (i, 0)),
            pl.BlockSpec((BLK, AW), lambda i: (i, 0)),
            pl.BlockSpec((BLK, BW), lambda i: (i, 0)),
        ] + _full_specs(consts),
        out_specs=[
            pl.BlockSpec((BLK, HID), lambda i: (i, 0)),
            pl.BlockSpec((BLK, PKW), lambda i: (i, 0)),
            pl.BlockSpec((BLK, PKW), lambda i: (i, 0)),
        ],
        out_shape=[
            jax.ShapeDtypeStruct((N, HID), jnp.float32),
            jax.ShapeDtypeStruct((N, PKW), jnp.float32),
            jax.ShapeDtypeStruct((N, PKW), jnp.float32),
        ],
    )
    return f(h, acca, accb, *consts)


EBLK_TC = 400
EBLK = E // EBLK_TC


def _edge_math_body(first, hsg_ref, hdg_ref, e_ref, a04, wedge, wea06, wee,
                    be, rep, pex, pma, pmb, va_ref, vb_ref, eo_ref):
    hsg = hsg_ref[...]
    hdg = hdg_ref[...]
    e = e_ref[...]
    if first:
        e = jnp.nan_to_num(e)
    he = jnp.dot(e, wedge[...], preferred_element_type=jnp.float32)
    m = hsg[:, :HID] + hdg[:, :HID] + he
    s = jnp.dot(jnp.abs(m), a04[...], preferred_element_type=jnp.float32)
    lin = (hsg[:, 22:26] + hdg[:, 22:26]
           + jnp.dot(e, wea06[...], preferred_element_type=jnp.float32))
    ex = jnp.exp(lin + s)
    exrep = jnp.dot(ex, rep[...], preferred_element_type=jnp.float32)
    msg = hsg[:, :HID] * exrep
    va_ref[...] = (jnp.dot(ex, pex[...], preferred_element_type=jnp.float32)
                   + jnp.dot(msg, pma[...],
                             preferred_element_type=jnp.float32))
    vb_ref[...] = jnp.dot(msg, pmb[...], preferred_element_type=jnp.float32)
    eo_ref[...] = jax.nn.relu(
        hsg[:, HID:HID + NEDGE] + hdg[:, HID:HID + NEDGE]
        + jnp.dot(e, wee[...], preferred_element_type=jnp.float32) + be[...])


def _edge_math(hsg, hdg, e, pc, first):
    pex = jnp.eye(HEADS, AW, dtype=jnp.float32)
    pma = jnp.zeros((HID, AW), jnp.float32).at[
        jnp.arange(AW - HEADS), jnp.arange(HEADS, AW)].set(1.0)
    pmb = jnp.zeros((HID, BW), jnp.float32).at[
        jnp.arange(AW - HEADS, HID), jnp.arange(BW)].set(1.0)
    consts = [pc['a04'], pc['wedge'], pc['wea06'], pc['wee'], _row(pc['be']),
              pc['rep'], pex, pma, pmb]
    f = pl.pallas_call(
        functools.partial(_edge_math_body, first),
        grid=(EBLK_TC,),
        in_specs=[
            pl.BlockSpec((EBLK, PKW), lambda i: (i, 0)),
            pl.BlockSpec((EBLK, PKW), lambda i: (i, 0)),
            pl.BlockSpec((EBLK, NEDGE), lambda i: (i, 0)),
        ] + _full_specs(consts),
        out_specs=[
            pl.BlockSpec((EBLK, AW), lambda i: (i, 0)),
            pl.BlockSpec((EBLK, BW), lambda i: (i, 0)),
            pl.BlockSpec((EBLK, NEDGE), lambda i: (i, 0)),
        ],
        out_shape=[
            jax.ShapeDtypeStruct((E, AW), jnp.float32),
            jax.ShapeDtypeStruct((E, BW), jnp.float32),
            jax.ShapeDtypeStruct((E, NEDGE), jnp.float32),
        ],
    )
    return f(hsg, hdg, e, *consts)


def _mlp(z, p):
    return jax.nn.relu(p['g'][...] * (jnp.dot(z, p['W'][...],
                                              preferred_element_type=jnp.float32)
                                      + p['b'][...]) + p['beta'][...])


def _readout_body(h4m_ref, acc5ma_ref, acc5mb_ref, prom_ref, bn,
                  w0, b0, g0, t0, w1, b1, g1, t1,
                  fcw0, fcb0, fcg0, fct0, fcw1, fcb1, fcg1, fct1,
                  fcw2, fcb2, fcg2, fct2, fcw3, fcb3, fcg3, fct3,
                  fcw4, fcb4, fcg4, fct4, wro, bro, out_ref):
    agg = _agg_from_acc(acc5ma_ref[...], acc5mb_ref[...])
    h5 = jax.nn.relu(h4m_ref[...] + agg + bn[...])
    p = jnp.nan_to_num(prom_ref[...])
    pp0 = _mlp(p, {'W': w0, 'b': b0, 'g': g0, 'beta': t0})
    pp = pp0 + _mlp(pp0, {'W': w1, 'b': b1, 'g': g1, 'beta': t1})
    z = h5 + pp
    z = _mlp(z, {'W': fcw0, 'b': fcb0, 'g': fcg0, 'beta': fct0})
    z = _mlp(z, {'W': fcw1, 'b': fcb1, 'g': fcg1, 'beta': fct1})
    z = _mlp(z, {'W': fcw2, 'b': fcb2, 'g': fcg2, 'beta': fct2})
    z = _mlp(z, {'W': fcw3, 'b': fcb3, 'g': fcg3, 'beta': fct3})
    z = _mlp(z, {'W': fcw4, 'b': fcb4, 'g': fcg4, 'beta': fct4})
    out_ref[...] = jnp.dot(z, wro[...],
                           preferred_element_type=jnp.float32) + bro[...]


def _readout(h4mid, acc5ma, acc5mb, prom_x, bnode, p0, p1, fc, ro):
    fcargs = []
    for lp in fc:
        fcargs += [lp['W'], _row(lp['b']), _row(lp['g']), _row(lp['beta'])]
    consts = ([_row(bnode),
               p0['W'], _row(p0['b']), _row(p0['g']), _row(p0['beta']),
               p1['W'], _row(p1['b']), _row(p1['g']), _row(p1['beta'])]
              + fcargs + [ro['W'], _row(ro['b'])])
    f = pl.pallas_call(
        _readout_body,
        grid=(1,),
        in_specs=[
            pl.BlockSpec((B, HID), lambda i: (0, 0)),
            pl.BlockSpec((B, AW), lambda i: (0, 0)),
            pl.BlockSpec((B, BW), lambda i: (0, 0)),
            pl.BlockSpec((B, NUMCHIP), lambda i: (0, 0)),
        ] + _full_specs(consts),
        out_specs=pl.BlockSpec((B, 3), lambda i: (0, 0)),
        out_shape=jax.ShapeDtypeStruct((B, 3), jnp.float32),
    )
    return f(h4mid, acc5ma, acc5mb, prom_x, *consts)


def _conv_consts(p):
    att = p['att']
    a_mat = jnp.zeros((HID, HEADS), jnp.float32)
    a_mat = a_mat.at[jnp.arange(HID), jnp.arange(HID) // DH].set(
        att.reshape(-1))
    we = p['We']
    z6 = jnp.zeros((HID, PKW - HID - NEDGE - HEADS), jnp.float32)
    wsx = jnp.concatenate(
        [p['W_src'], we[:HID], 0.6 * jnp.dot(p['W_src'], a_mat), z6], axis=1)
    wdx = jnp.concatenate(
        [p['W_dst'], we[HID:2 * HID], 0.6 * jnp.dot(p['W_dst'], a_mat), z6],
        axis=1)
    return {
        'wsx': wsx, 'wdx': wdx,
        'wee': we[2 * HID:], 'be': p['be'],
        'wedge': p['W_edge'],
        'wea06': 0.6 * jnp.dot(p['W_edge'], a_mat),
        'a04': 0.4 * a_mat,
        'rep': jnp.zeros((HEADS, HID), jnp.float32).at[
            jnp.arange(HID) // DH, jnp.arange(HID)].set(1.0),
        'bnode': p['b_node'],
    }


_USE_SC_SCATTER = False


def kernel(x, prom_x, edge_attr, edge_index, batch, params):
    src = edge_index[0]
    dst = edge_index[1]
    prom_x = prom_x.reshape(-1, NUMCHIP)
    penc = _conv_consts(params['enc'])
    pdec = _conv_consts(params['encdec'])
    pe = _pe_table()
    zeros_half = jnp.zeros((SHROWS, AW), jnp.float32)
    zeros_b = jnp.zeros((SHROWS, BW), jnp.float32)

    def cat_acc(a0, a1):
        return jnp.concatenate([a0[:HALF], a1[:HALF]], axis=0)

    h, hsx, hdx = _front(x, pe, params['emb0'], params['emb1'], penc)
    e = edge_attr
    bnode_prev = None
    acca = accb = None
    for layer in range(5):
        pc = penc if layer == 0 else pdec
        if layer > 0:
            h, hsx, hdx = _tca(h, acca, accb, bnode_prev, pc)
        hsg, hdg = _sc_gather(hsx, hdx, src, dst)
        va, vb, e = _edge_math(hsg, hdg, e, pc, first=(layer == 0))
        if _USE_SC_SCATTER:
            aa0, aa1, ab0, ab1 = _sc_scatter(va, vb, dst, zeros_half,
                                             zeros_b)
            acca, accb = cat_acc(aa0, aa1), cat_acc(ab0, ab1)
        else:
            acca = jax.ops.segment_sum(va, dst, num_segments=N)
            accb = jax.ops.segment_sum(vb, dst, num_segments=N)
        bnode_prev = pc['bnode']

    h4mid = h.reshape(B, L, HID)[:, L // 2, :]
    acc5ma = acca.reshape(B, L, AW)[:, L // 2, :]
    acc5mb = accb.reshape(B, L, BW)[:, L // 2, :]
    return _readout(h4mid, acc5ma, acc5mb, prom_x, bnode_prev,
                    params['emb0'], params['emb1'], params['fc'],
                    params['readout'])
